# baseline (speedup 1.0000x reference)
import dataclasses
import functools

import jax
import jax.numpy as jnp
from jax import lax
from jax.experimental import pallas as pl
from jax.experimental.pallas import tpu as pltpu
from jax.experimental.pallas import tpu_sc as plsc

B = 16384
U_DIM = 64
I_DIM = 64
N_ACT = 100
VOCAB = 100000
Y_LOW = 0.0
Y_HIGH = 5.5

NC = 2
NS = 16
NW = NC * NS
BPW = B // NW
VL = 16


def _gather_sc(table, x, col):
    mesh = plsc.VectorSubcoreMesh(core_axis_name="c", subcore_axis_name="s")
    cp = pltpu.CompilerParams()
    if "needs_layout_passes" in pltpu.CompilerParams.__dataclass_fields__:
        cp = dataclasses.replace(cp, needs_layout_passes=False)

    @functools.partial(
        pl.kernel,
        mesh=mesh,
        compiler_params=cp,
        out_type=jax.ShapeDtypeStruct((B, 128), jnp.float32),
        scratch_types=[
            pltpu.VMEM((BPW, 2), jnp.int32),
            pltpu.VMEM((BPW,), jnp.int32),
            pltpu.VMEM((BPW // 2, 128), jnp.float32),
            pltpu.SemaphoreType.DMA,
        ],
    )
    def k(t_hbm, x_hbm, g_hbm, x_v, j_v, rows_v, sem):
        wid = lax.axis_index("s") * NC + lax.axis_index("c")
        base = wid * BPW
        pltpu.sync_copy(x_hbm.at[pl.ds(base, BPW)], x_v)

        cols = jnp.zeros((VL,), jnp.int32) + col
        riota = lax.iota(jnp.int32, VL)

        @pl.loop(0, BPW, step=VL)
        def _(j):
            v = plsc.load_gather(x_v, [riota + j, cols])
            j_v[pl.ds(j, VL)] = lax.shift_right_logical(v, 1)

        half = BPW // 2
        for c in range(2):
            pltpu.async_copy(
                t_hbm.at[j_v.at[pl.ds(c * half, half)]], rows_v, sem).wait()
            pltpu.sync_copy(rows_v, g_hbm.at[pl.ds(base + c * half, half)])

    return k(table, x)


def _mlp_body(gu_ref, gi_ref, x_ref, w1u_ref, w1i_ref, b1_ref,
              w2_ref, b2_ref, o_ref):
    xb = x_ref[...]
    pu = (xb[:, 0:1] & 1) > 0
    pi = (xb[:, 1:2] & 1) > 0
    gu = gu_ref[...]
    gi = gi_ref[...]
    u = jnp.where(pu, gu[:, U_DIM:], gu[:, :U_DIM])
    i = jnp.where(pi, gi[:, U_DIM:], gi[:, :U_DIM])
    h = jnp.dot(u, w1u_ref[...], preferred_element_type=jnp.float32)
    h += jnp.dot(i, w1i_ref[...], preferred_element_type=jnp.float32)
    h = jnp.maximum(h + b1_ref[...], 0.0)
    out = jnp.dot(h, w2_ref[...], preferred_element_type=jnp.float32)
    out += b2_ref[...]
    o_ref[...] = jax.nn.sigmoid(out) * (Y_HIGH - Y_LOW) + Y_LOW


def _mlp_tc(gu, gi, x, W1, b1, W2, b2):
    BM = 2048
    grid = (B // BM,)
    w1u = W1[:U_DIM]
    w1i = W1[U_DIM:]
    b1r = b1.reshape(1, N_ACT)
    b2r = b2.reshape(1, 1)
    return pl.pallas_call(
        _mlp_body,
        grid=grid,
        in_specs=[
            pl.BlockSpec((BM, 128), lambda m: (m, 0)),
            pl.BlockSpec((BM, 128), lambda m: (m, 0)),
            pl.BlockSpec((BM, 2), lambda m: (m, 0)),
            pl.BlockSpec((U_DIM, N_ACT), lambda m: (0, 0)),
            pl.BlockSpec((I_DIM, N_ACT), lambda m: (0, 0)),
            pl.BlockSpec((1, N_ACT), lambda m: (0, 0)),
            pl.BlockSpec((N_ACT, 1), lambda m: (0, 0)),
            pl.BlockSpec((1, 1), lambda m: (0, 0)),
        ],
        out_specs=pl.BlockSpec((BM, 1), lambda m: (m, 0)),
        out_shape=jax.ShapeDtypeStruct((B, 1), jnp.float32),
    )(gu, gi, x, w1u, w1i, b1r, W2, b2r)


@jax.jit
def kernel(x, user_table, item_table, W1, b1, W2, b2):
    uc = jnp.concatenate(
        [lax.slice(user_table, (0, 0), (VOCAB, 64), (2, 1)),
         lax.slice(user_table, (1, 0), (VOCAB, 64), (2, 1))], axis=1)
    ic = jnp.concatenate(
        [lax.slice(item_table, (0, 0), (VOCAB, 64), (2, 1)),
         lax.slice(item_table, (1, 0), (VOCAB, 64), (2, 1))], axis=1)
    gu = _gather_sc(uc, x, 0)
    gi = _gather_sc(ic, x, 1)
    return _mlp_tc(gu, gi, x, W1, b1, W2, b2)


# speedup vs baseline: 9.4456x; 9.4456x over previous
import dataclasses
import functools

import jax
import jax.numpy as jnp
from jax import lax
from jax.experimental import pallas as pl
from jax.experimental.pallas import tpu as pltpu
from jax.experimental.pallas import tpu_sc as plsc

B = 16384
U_DIM = 64
I_DIM = 64
N_ACT = 100
VOCAB = 100000
Y_LOW = 0.0
Y_HIGH = 5.5

NC = 2
NS = 16
NW = NC * NS
BPW = B // NW
VL = 16


def _gather_sc(table, x, col):
    mesh = plsc.VectorSubcoreMesh(core_axis_name="c", subcore_axis_name="s")
    cp = pltpu.CompilerParams()
    if "needs_layout_passes" in pltpu.CompilerParams.__dataclass_fields__:
        cp = dataclasses.replace(cp, needs_layout_passes=False)

    @functools.partial(
        pl.kernel,
        mesh=mesh,
        compiler_params=cp,
        out_type=jax.ShapeDtypeStruct((B, 128), jnp.float32),
        scratch_types=[
            pltpu.VMEM((BPW, 2), jnp.int32),
            pltpu.VMEM((BPW,), jnp.int32),
            pltpu.VMEM((BPW // 2, 128), jnp.float32),
            pltpu.SemaphoreType.DMA,
        ],
    )
    def k(t_hbm, x_hbm, g_hbm, x_v, j_v, rows_v, sem):
        wid = lax.axis_index("s") * NC + lax.axis_index("c")
        base = wid * BPW
        pltpu.sync_copy(x_hbm.at[pl.ds(base, BPW)], x_v)

        cols = jnp.zeros((VL,), jnp.int32) + col
        riota = lax.iota(jnp.int32, VL)

        @pl.loop(0, BPW, step=VL)
        def _(j):
            v = plsc.load_gather(x_v, [riota + j, cols])
            j_v[pl.ds(j, VL)] = lax.shift_right_logical(v, 1)

        half = BPW // 2
        for c in range(2):
            pltpu.async_copy(
                t_hbm.at[j_v.at[pl.ds(c * half, half)]], rows_v, sem).wait()
            pltpu.sync_copy(rows_v, g_hbm.at[pl.ds(base + c * half, half)])

    return k(table, x)


def _mlp_body(gu_ref, gi_ref, x_ref, w1u_ref, w1i_ref, b1_ref,
              w2_ref, b2_ref, o_ref):
    xb = x_ref[...]
    pu = (xb[:, 0:1] & 1) > 0
    pi = (xb[:, 1:2] & 1) > 0
    gu = gu_ref[...]
    gi = gi_ref[...]
    u = jnp.where(pu, gu[:, U_DIM:], gu[:, :U_DIM])
    i = jnp.where(pi, gi[:, U_DIM:], gi[:, :U_DIM])
    h = jnp.dot(u, w1u_ref[...], preferred_element_type=jnp.float32)
    h += jnp.dot(i, w1i_ref[...], preferred_element_type=jnp.float32)
    h = jnp.maximum(h + b1_ref[...], 0.0)
    out = jnp.dot(h, w2_ref[...], preferred_element_type=jnp.float32)
    out += b2_ref[...]
    o_ref[...] = jax.nn.sigmoid(out) * (Y_HIGH - Y_LOW) + Y_LOW


def _mlp_tc(gu, gi, x, W1, b1, W2, b2):
    BM = 8192
    grid = (B // BM,)
    w1u = W1[:U_DIM]
    w1i = W1[U_DIM:]
    b1r = b1.reshape(1, N_ACT)
    b2r = b2.reshape(1, 1)
    return pl.pallas_call(
        _mlp_body,
        grid=grid,
        in_specs=[
            pl.BlockSpec((BM, 128), lambda m: (m, 0)),
            pl.BlockSpec((BM, 128), lambda m: (m, 0)),
            pl.BlockSpec((BM, 2), lambda m: (m, 0)),
            pl.BlockSpec((U_DIM, N_ACT), lambda m: (0, 0)),
            pl.BlockSpec((I_DIM, N_ACT), lambda m: (0, 0)),
            pl.BlockSpec((1, N_ACT), lambda m: (0, 0)),
            pl.BlockSpec((N_ACT, 1), lambda m: (0, 0)),
            pl.BlockSpec((1, 1), lambda m: (0, 0)),
        ],
        out_specs=pl.BlockSpec((BM, 1), lambda m: (m, 0)),
        out_shape=jax.ShapeDtypeStruct((B, 1), jnp.float32),
    )(gu, gi, x, w1u, w1i, b1r, W2, b2r)


@jax.jit
def kernel(x, user_table, item_table, W1, b1, W2, b2):
    uc = user_table[:VOCAB].reshape(VOCAB // 2, 128)
    ic = item_table.reshape(VOCAB // 2, 128)
    gu = _gather_sc(uc, x, 0)
    gi = _gather_sc(ic, x, 1)
    return _mlp_tc(gu, gi, x, W1, b1, W2, b2)


# speedup vs baseline: 9.6710x; 1.0239x over previous
import dataclasses
import functools

import jax
import jax.numpy as jnp
from jax import lax
from jax.experimental import pallas as pl
from jax.experimental.pallas import tpu as pltpu
from jax.experimental.pallas import tpu_sc as plsc

B = 16384
U_DIM = 64
I_DIM = 64
N_ACT = 100
VOCAB = 100000
Y_LOW = 0.0
Y_HIGH = 5.5

NC = 2
NS = 16
NW = NC * NS
BPW = B // NW
VL = 16


def _gather_sc(table, x, col):
    mesh = plsc.VectorSubcoreMesh(core_axis_name="c", subcore_axis_name="s")
    cp = pltpu.CompilerParams()
    if "needs_layout_passes" in pltpu.CompilerParams.__dataclass_fields__:
        cp = dataclasses.replace(cp, needs_layout_passes=False)

    @functools.partial(
        pl.kernel,
        mesh=mesh,
        compiler_params=cp,
        out_type=jax.ShapeDtypeStruct((B, 128), jnp.float32),
        scratch_types=[
            pltpu.VMEM((2 * BPW // 128, 128), jnp.int32),
            pltpu.VMEM((BPW,), jnp.int32),
            pltpu.VMEM((BPW // 2, 128), jnp.float32),
            pltpu.SemaphoreType.DMA,
        ],
    )
    def k(t_hbm, x_hbm, g_hbm, x_v, j_v, rows_v, sem):
        wid = lax.axis_index("s") * NC + lax.axis_index("c")
        base = wid * BPW
        xrows = 2 * BPW // 128
        pltpu.sync_copy(x_hbm.at[pl.ds(wid * xrows, xrows)], x_v)

        riota = lax.iota(jnp.int32, VL)

        @pl.loop(0, BPW, step=VL)
        def _(j):
            flat = (riota + j) * 2 + col
            v = plsc.load_gather(
                x_v, [lax.shift_right_logical(flat, 7), flat & 127])
            j_v[pl.ds(j, VL)] = lax.shift_right_logical(v, 1)

        half = BPW // 2
        for c in range(2):
            pltpu.async_copy(
                t_hbm.at[j_v.at[pl.ds(c * half, half)]], rows_v, sem).wait()
            pltpu.sync_copy(rows_v, g_hbm.at[pl.ds(base + c * half, half)])

    return k(table, x)


def _mlp_body(gu_ref, gi_ref, x_ref, w1u_ref, w1i_ref, b1_ref,
              w2_ref, b2_ref, o_ref):
    xb = x_ref[...]
    pu = (xb[:, 0:1] & 1) > 0
    pi = (xb[:, 1:2] & 1) > 0
    gu = gu_ref[...]
    gi = gi_ref[...]
    u = jnp.where(pu, gu[:, U_DIM:], gu[:, :U_DIM])
    i = jnp.where(pi, gi[:, U_DIM:], gi[:, :U_DIM])
    h = jnp.dot(u, w1u_ref[...], preferred_element_type=jnp.float32)
    h += jnp.dot(i, w1i_ref[...], preferred_element_type=jnp.float32)
    h = jnp.maximum(h + b1_ref[...], 0.0)
    out = jnp.dot(h, w2_ref[...], preferred_element_type=jnp.float32)
    out += b2_ref[...]
    o_ref[...] = jax.nn.sigmoid(out) * (Y_HIGH - Y_LOW) + Y_LOW


def _mlp_tc(gu, gi, x, W1, b1, W2, b2):
    BM = 2048
    grid = (B // BM,)
    w1u = W1[:U_DIM]
    w1i = W1[U_DIM:]
    b1r = b1.reshape(1, N_ACT)
    b2r = b2.reshape(1, 1)
    return pl.pallas_call(
        _mlp_body,
        grid=grid,
        in_specs=[
            pl.BlockSpec((BM, 128), lambda m: (m, 0)),
            pl.BlockSpec((BM, 128), lambda m: (m, 0)),
            pl.BlockSpec((BM, 2), lambda m: (m, 0)),
            pl.BlockSpec((U_DIM, N_ACT), lambda m: (0, 0)),
            pl.BlockSpec((I_DIM, N_ACT), lambda m: (0, 0)),
            pl.BlockSpec((1, N_ACT), lambda m: (0, 0)),
            pl.BlockSpec((N_ACT, 1), lambda m: (0, 0)),
            pl.BlockSpec((1, 1), lambda m: (0, 0)),
        ],
        out_specs=pl.BlockSpec((BM, 1), lambda m: (m, 0)),
        out_shape=jax.ShapeDtypeStruct((B, 1), jnp.float32),
    )(gu, gi, x, w1u, w1i, b1r, W2, b2r)


@jax.jit
def kernel(x, user_table, item_table, W1, b1, W2, b2):
    uc = user_table[:VOCAB].reshape(VOCAB // 2, 128)
    ic = item_table.reshape(VOCAB // 2, 128)
    x_r = x.reshape(2 * B // 128, 128)
    gu = _gather_sc(uc, x_r, 0)
    gi = _gather_sc(ic, x_r, 1)
    return _mlp_tc(gu, gi, x, W1, b1, W2, b2)


# speedup vs baseline: 10.5536x; 1.0913x over previous
import dataclasses
import functools

import jax
import jax.numpy as jnp
from jax import lax
from jax.experimental import pallas as pl
from jax.experimental.pallas import tpu as pltpu
from jax.experimental.pallas import tpu_sc as plsc

B = 16384
U_DIM = 64
I_DIM = 64
N_ACT = 100
VOCAB = 100000
Y_LOW = 0.0
Y_HIGH = 5.5

NC = 2
NS = 16
NW = NC * NS
BPW = B // NW
VL = 16


def _gather_sc(table, x, col):
    mesh = plsc.VectorSubcoreMesh(core_axis_name="c", subcore_axis_name="s")
    cp = pltpu.CompilerParams()
    if "needs_layout_passes" in pltpu.CompilerParams.__dataclass_fields__:
        cp = dataclasses.replace(cp, needs_layout_passes=False)

    @functools.partial(
        pl.kernel,
        mesh=mesh,
        compiler_params=cp,
        out_type=jax.ShapeDtypeStruct((B, 128), jnp.float32),
        scratch_types=[
            pltpu.VMEM((2 * BPW // 128, 128), jnp.int32),
            pltpu.VMEM((BPW,), jnp.int32),
            pltpu.VMEM((BPW // 2, 128), jnp.float32),
            pltpu.SemaphoreType.DMA,
        ],
    )
    def k(t_hbm, x_hbm, g_hbm, x_v, j_v, rows_v, sem):
        wid = lax.axis_index("s") * NC + lax.axis_index("c")
        base = wid * BPW
        xrows = 2 * BPW // 128
        pltpu.sync_copy(x_hbm.at[pl.ds(wid * xrows, xrows)], x_v)

        riota = lax.iota(jnp.int32, VL)

        @pl.loop(0, BPW, step=VL)
        def _(j):
            flat = (riota + j) * 2 + col
            v = plsc.load_gather(
                x_v, [lax.shift_right_logical(flat, 7), flat & 127])
            j_v[pl.ds(j, VL)] = v

        half = BPW // 2
        for c in range(2):
            pltpu.async_copy(
                t_hbm.at[j_v.at[pl.ds(c * half, half)]], rows_v, sem).wait()
            pltpu.sync_copy(rows_v, g_hbm.at[pl.ds(base + c * half, half)])

    return k(table, x)


def _mlp_body(gu_ref, gi_ref, w1u_ref, w1i_ref, b1_ref,
              w2_ref, b2_ref, o_ref):
    u = gu_ref[:, :U_DIM]
    i = gi_ref[:, U_DIM:]
    h = jnp.dot(u, w1u_ref[...], preferred_element_type=jnp.float32)
    h += jnp.dot(i, w1i_ref[...], preferred_element_type=jnp.float32)
    h = jnp.maximum(h + b1_ref[...], 0.0)
    out = jnp.dot(h, w2_ref[...], preferred_element_type=jnp.float32)
    out += b2_ref[...]
    o_ref[...] = jax.nn.sigmoid(out) * (Y_HIGH - Y_LOW) + Y_LOW


def _mlp_tc(gu, gi, W1, b1, W2, b2):
    BM = 2048
    grid = (B // BM,)
    w1u = W1[:U_DIM]
    w1i = W1[U_DIM:]
    b1r = b1.reshape(1, N_ACT)
    b2r = b2.reshape(1, 1)
    return pl.pallas_call(
        _mlp_body,
        grid=grid,
        in_specs=[
            pl.BlockSpec((BM, 128), lambda m: (m, 0)),
            pl.BlockSpec((BM, 128), lambda m: (m, 0)),
            pl.BlockSpec((U_DIM, N_ACT), lambda m: (0, 0)),
            pl.BlockSpec((I_DIM, N_ACT), lambda m: (0, 0)),
            pl.BlockSpec((1, N_ACT), lambda m: (0, 0)),
            pl.BlockSpec((N_ACT, 1), lambda m: (0, 0)),
            pl.BlockSpec((1, 1), lambda m: (0, 0)),
        ],
        out_specs=pl.BlockSpec((BM, 1), lambda m: (m, 0)),
        out_shape=jax.ShapeDtypeStruct((B, 1), jnp.float32),
    )(gu, gi, w1u, w1i, b1r, W2, b2r)


@jax.jit
def kernel(x, user_table, item_table, W1, b1, W2, b2):
    uc = jnp.pad(user_table[:VOCAB], ((0, 0), (0, 64)))
    ic = jnp.pad(item_table, ((0, 0), (64, 0)))
    x_r = x.reshape(2 * B // 128, 128)
    gu = _gather_sc(uc, x_r, 0)
    gi = _gather_sc(ic, x_r, 1)
    return _mlp_tc(gu, gi, W1, b1, W2, b2)
